# baseline (speedup 1.0000x reference)
import math

import jax
import jax.numpy as jnp
from jax.experimental import pallas as pl
from jax.experimental.pallas import tpu as pltpu

B, H, W = 4, 48, 48
HW = H * W
RADIUS = 6.0
EPS = 1e-5


def _params_body(x_ref, mw_ref, mb_ref, nw_ref, nb_ref, cw_ref, cb_ref,
                 mean_ref, det_ref, park_ref):
    x2 = x_ref[...].reshape(B * HW, x_ref.shape[-1])
    tt = jnp.tanh(
        jnp.dot(x2, mw_ref[...], preferred_element_type=jnp.float32)
        + mb_ref[...])
    mo = (jnp.dot(tt, nw_ref[...], preferred_element_type=jnp.float32)
          + nb_ref[...])
    xc = (jnp.dot(tt, cw_ref[...], preferred_element_type=jnp.float32)
          + cb_ref[...])
    xc3 = xc.reshape(B, HW, 2)
    m = jnp.mean(xc3, axis=(1, 2), keepdims=True)
    v = jnp.mean((xc3 - m) ** 2, axis=(1, 2), keepdims=True)
    xn = (xc3 - m) / jnp.sqrt(v + EPS)
    s = jax.nn.sigmoid(xn) * 5.0 + 0.05
    cx = s[:, :, 0]
    cy = s[:, :, 1]
    det = cx * cy
    det_ref[...] = det
    inv_denom = (1.0 / 6.28) * jax.lax.rsqrt(det)

    mo4 = mo.reshape(B, H, W, 2)
    lane = jax.lax.broadcasted_iota(jnp.int32, (B, H, W, 2), 3)
    col = jax.lax.broadcasted_iota(
        jnp.int32, (B, H, W, 2), 2).astype(jnp.float32)
    row = jax.lax.broadcasted_iota(
        jnp.int32, (B, H, W, 2), 1).astype(jnp.float32)
    coord = jnp.where(lane == 0, col, row)
    mean_ref[...] = coord + mo4

    mx = coord[..., 0].reshape(B, HW) + mo.reshape(B, HW, 2)[:, :, 0]
    my = coord[..., 1].reshape(B, HW) + mo.reshape(B, HW, 2)[:, :, 1]
    nicx = -0.5 / cx
    nicy = -0.5 / cy
    z = jnp.zeros_like(mx)
    park = jnp.stack([mx, my, nicx, nicy, inv_denom, z, z, z], axis=-1)
    park_ref[...] = park.reshape(B, H, W, 8)


def _mask_body(park_ref, corr_ref, out_ref):
    p = park_ref[0]
    mx = p[:, :, 0:1]
    my = p[:, :, 1:2]
    nicx = p[:, :, 2:3]
    nicy = p[:, :, 3:4]
    ind = p[:, :, 4:5]
    bi = p.shape[0]
    t = jax.lax.broadcasted_iota(
        jnp.int32, (bi, W, W), 2).astype(jnp.float32)
    dx = t - mx
    dy = t - my
    a = jnp.exp(nicy * dy * dy) * (jnp.abs(dy) <= RADIUS)
    b = (jnp.exp(nicx * dx * dx) * (jnp.abs(dx) <= RADIUS)
         * ind)
    cr = corr_ref[0]
    out_ref[0] = cr + cr * (a[:, :, :, None] * b[:, :, None, :])


def kernel(x, corr, map_w, map_b, mean_w, mean_b, cov_w, cov_b):
    mean, det, park = pl.pallas_call(
        _params_body,
        out_shape=(
            jax.ShapeDtypeStruct((B, H, W, 2), jnp.float32),
            jax.ShapeDtypeStruct((B, HW), jnp.float32),
            jax.ShapeDtypeStruct((B, H, W, 8), jnp.float32),
        ),
    )(x, map_w.T, map_b.reshape(1, -1), mean_w.T, mean_b.reshape(1, -1),
      cov_w.T, cov_b.reshape(1, -1))

    corr1 = pl.pallas_call(
        _mask_body,
        grid=(B, H // 4),
        in_specs=[
            pl.BlockSpec((1, 4, W, 8), lambda b, i: (b, i, 0, 0)),
            pl.BlockSpec((1, 4, W, H, W), lambda b, i: (b, i, 0, 0, 0)),
        ],
        out_specs=pl.BlockSpec((1, 4, W, H, W), lambda b, i: (b, i, 0, 0, 0)),
        out_shape=jax.ShapeDtypeStruct((B, H, W, H, W), jnp.float32),
        compiler_params=pltpu.CompilerParams(
            dimension_semantics=("parallel", "parallel")),
    )(park, corr)

    return (corr1, mean, det)


# speedup vs baseline: 1.0141x; 1.0141x over previous
import math

import jax
import jax.numpy as jnp
from jax.experimental import pallas as pl
from jax.experimental.pallas import tpu as pltpu

B, H, W = 4, 48, 48
HW = H * W
RADIUS = 6.0
EPS = 1e-5


def _params_body(x_ref, mw_ref, mb_ref, nw_ref, nb_ref, cw_ref, cb_ref,
                 mean_ref, det_ref, park_ref):
    x2 = x_ref[...].reshape(B * HW, x_ref.shape[-1])
    tt = jnp.tanh(
        jnp.dot(x2, mw_ref[...], preferred_element_type=jnp.float32)
        + mb_ref[...])
    mo = (jnp.dot(tt, nw_ref[...], preferred_element_type=jnp.float32)
          + nb_ref[...])
    xc = (jnp.dot(tt, cw_ref[...], preferred_element_type=jnp.float32)
          + cb_ref[...])
    xc3 = xc.reshape(B, HW, 2)
    m = jnp.mean(xc3, axis=(1, 2), keepdims=True)
    v = jnp.mean((xc3 - m) ** 2, axis=(1, 2), keepdims=True)
    xn = (xc3 - m) / jnp.sqrt(v + EPS)
    s = jax.nn.sigmoid(xn) * 5.0 + 0.05
    cx = s[:, :, 0]
    cy = s[:, :, 1]
    det = cx * cy
    det_ref[...] = det
    inv_denom = (1.0 / 6.28) * jax.lax.rsqrt(det)

    mo4 = mo.reshape(B, H, W, 2)
    lane = jax.lax.broadcasted_iota(jnp.int32, (B, H, W, 2), 3)
    col = jax.lax.broadcasted_iota(
        jnp.int32, (B, H, W, 2), 2).astype(jnp.float32)
    row = jax.lax.broadcasted_iota(
        jnp.int32, (B, H, W, 2), 1).astype(jnp.float32)
    coord = jnp.where(lane == 0, col, row)
    mean_ref[...] = coord + mo4

    mx = coord[..., 0].reshape(B, HW) + mo.reshape(B, HW, 2)[:, :, 0]
    my = coord[..., 1].reshape(B, HW) + mo.reshape(B, HW, 2)[:, :, 1]
    nicx = -0.5 / cx
    nicy = -0.5 / cy
    z = jnp.zeros_like(mx)
    park = jnp.stack([mx, my, nicx, nicy, inv_denom, z, z, z], axis=-1)
    park_ref[...] = park.reshape(B, H, W, 8)


def _mask_body(park_ref, corr_ref, out_ref):
    p = park_ref[0]
    mx = p[:, :, 0:1]
    my = p[:, :, 1:2]
    nicx = p[:, :, 2:3]
    nicy = p[:, :, 3:4]
    ind = p[:, :, 4:5]
    bi = p.shape[0]
    t = jax.lax.broadcasted_iota(
        jnp.int32, (bi, W, W), 2).astype(jnp.float32)
    dx = t - mx
    dy = t - my
    a = jnp.exp(nicy * dy * dy) * (jnp.abs(dy) <= RADIUS)
    b = (jnp.exp(nicx * dx * dx) * (jnp.abs(dx) <= RADIUS)
         * ind)
    cr = corr_ref[0]
    out_ref[0] = cr + cr * (a[:, :, :, None] * b[:, :, None, :])


def kernel(x, corr, map_w, map_b, mean_w, mean_b, cov_w, cov_b):
    mean, det, park = pl.pallas_call(
        _params_body,
        out_shape=(
            jax.ShapeDtypeStruct((B, H, W, 2), jnp.float32),
            jax.ShapeDtypeStruct((B, HW), jnp.float32),
            jax.ShapeDtypeStruct((B, H, W, 8), jnp.float32),
        ),
    )(x, map_w.T, map_b.reshape(1, -1), mean_w.T, mean_b.reshape(1, -1),
      cov_w.T, cov_b.reshape(1, -1))

    corr1 = pl.pallas_call(
        _mask_body,
        grid=(B, H // 8),
        in_specs=[
            pl.BlockSpec((1, 8, W, 8), lambda b, i: (b, i, 0, 0)),
            pl.BlockSpec((1, 8, W, H, W), lambda b, i: (b, i, 0, 0, 0)),
        ],
        out_specs=pl.BlockSpec((1, 8, W, H, W), lambda b, i: (b, i, 0, 0, 0)),
        out_shape=jax.ShapeDtypeStruct((B, H, W, H, W), jnp.float32),
        compiler_params=pltpu.CompilerParams(
            dimension_semantics=("parallel", "parallel")),
    )(park, corr)

    return (corr1, mean, det)


# speedup vs baseline: 1.0784x; 1.0634x over previous
import jax
import jax.numpy as jnp
from jax.experimental import pallas as pl
from jax.experimental.pallas import tpu as pltpu

B, H, W = 4, 48, 48
HW = H * W
N = B * HW
BI = 8
RADIUS = 6.0
EPS = 1e-5


def _params_body(x_ref, mw_ref, mb_ref, hw_ref, hb_ref, bm_ref, bmt_ref,
                 col_ref, row_ref, mean_ref, det_ref, park_ref):
    x2 = x_ref[...].reshape(N, x_ref.shape[-1])
    tt = jnp.tanh(
        jnp.dot(x2, mw_ref[...], preferred_element_type=jnp.float32)
        + mb_ref[...])
    ht = jax.lax.dot_general(
        hw_ref[...], tt, (((1,), (1,)), ((), ())),
        preferred_element_type=jnp.float32) + hb_ref[...]
    xc = ht[2:4]
    inv = 1.0 / (2.0 * HW)
    m = jnp.sum(jnp.dot(xc, bmt_ref[...],
                        preferred_element_type=jnp.float32),
                axis=0, keepdims=True) * inv
    mf = jnp.dot(m, bm_ref[...],
                 preferred_element_type=jnp.float32)
    d = xc - mf
    vs = jnp.sum(jnp.dot(d * d, bmt_ref[...],
                         preferred_element_type=jnp.float32),
                 axis=0, keepdims=True) * inv
    rs = jax.lax.rsqrt(vs + EPS)
    rsf = jnp.dot(rs, bm_ref[...],
                  preferred_element_type=jnp.float32)
    s = jax.nn.sigmoid(d * rsf) * 5.0 + 0.05
    cx = s[0:1]
    cy = s[1:2]
    det = cx * cy
    det_ref[...] = det.reshape(B, HW)
    invd = (1.0 / 6.28) * jax.lax.rsqrt(det)
    mx = ht[0:1] + col_ref[...]
    my = ht[1:2] + row_ref[...]
    mean_ref[...] = jnp.concatenate([mx, my], axis=0).T.reshape(B, H, W, 2)
    park_ref[...] = jnp.concatenate(
        [mx, my, -0.5 / cx, -0.5 / cy, invd, det, det, det], axis=0)


def _mask_body(park_ref, corr_ref, out_ref):
    pt = park_ref[...].T

    def prow(k):
        return pt[:, k:k + 1].reshape(BI, W, 1)

    mx = prow(0)
    my = prow(1)
    nicx = prow(2)
    nicy = prow(3)
    ind = prow(4)
    t = jax.lax.broadcasted_iota(
        jnp.int32, (BI, W, W), 2).astype(jnp.float32)
    dx = t - mx
    dy = t - my
    a = jnp.exp(nicy * dy * dy) * (jnp.abs(dy) <= RADIUS)
    b = (jnp.exp(nicx * dx * dx) * (jnp.abs(dx) <= RADIUS)
         * ind)
    cr = corr_ref[0]
    out_ref[0] = cr + cr * (a[:, :, :, None] * b[:, :, None, :])


def kernel(x, corr, map_w, map_b, mean_w, mean_b, cov_w, cov_b):
    f32 = jnp.float32
    hw8 = jnp.concatenate(
        [mean_w, cov_w, jnp.zeros((4, mean_w.shape[1]), f32)], axis=0)
    hb8 = jnp.concatenate(
        [mean_b, cov_b, jnp.zeros((4,), f32)]).reshape(8, 1)
    bm = jnp.repeat(jnp.eye(B, dtype=f32), HW, axis=1)
    pix = jnp.arange(N, dtype=jnp.int32)
    col = (pix % W).astype(f32).reshape(1, N)
    row = ((pix // W) % H).astype(f32).reshape(1, N)

    mean, det, park2 = pl.pallas_call(
        _params_body,
        out_shape=(
            jax.ShapeDtypeStruct((B, H, W, 2), f32),
            jax.ShapeDtypeStruct((B, HW), f32),
            jax.ShapeDtypeStruct((8, N), f32),
        ),
    )(x, map_w.T, map_b.reshape(1, -1), hw8, hb8, bm, bm.T, col, row)

    nb = H // BI
    corr1 = pl.pallas_call(
        _mask_body,
        grid=(B, nb),
        in_specs=[
            pl.BlockSpec((8, BI * W), lambda b, i: (0, b * nb + i)),
            pl.BlockSpec((1, BI, W, H, W), lambda b, i: (b, i, 0, 0, 0)),
        ],
        out_specs=pl.BlockSpec(
            (1, BI, W, H, W), lambda b, i: (b, i, 0, 0, 0)),
        out_shape=jax.ShapeDtypeStruct((B, H, W, H, W), f32),
        compiler_params=pltpu.CompilerParams(
            dimension_semantics=("parallel", "parallel")),
    )(park2, corr)

    return (corr1, mean, det)
